# baseline (speedup 1.0000x reference)
import functools

import jax
import jax.numpy as jnp
from jax.experimental import pallas as pl
from jax.experimental.pallas import tpu as pltpu

EPS = 1e-5
LANE = 128
WPAD = 64
COL0 = 16


def _build_slab_and_matmul(pad_ref, slab_ref, w_ref, hp, wp):
    for dh in range(3):
        for dw in range(3):
            t = dh * 3 + dw
            slab_ref[:, 0:wp, t * LANE:(t + 1) * LANE] = (
                pad_ref[dh:dh + hp, COL0 - 1 + dw:COL0 - 1 + dw + wp, :])
    return jnp.dot(slab_ref[...].reshape(hp * WPAD, 9 * LANE), w_ref[...],
                   preferred_element_type=jnp.float32)


def _scale_shift(st_ref, g_ref, b_ref, count):
    s = st_ref[0, :]
    sq = st_ref[1, :]
    mean = s / count
    var = jnp.maximum(sq / count - mean * mean, 0.0)
    scale = g_ref[0] * jax.lax.rsqrt(var + EPS)
    shift = b_ref[0] - mean * scale
    return scale, shift


def _down_kernel(count, x_ref, w1_ref, w2_ref, g1_ref, b1_ref, g2_ref,
                 b2_ref, out_ref, xs_ref, y_ref, pad_ref, slab_ref,
                 st1_ref, st2_ref):
    p = pl.program_id(0)
    i = pl.program_id(1)
    cin = x_ref.shape[1]
    H = xs_ref.shape[0]
    W = xs_ref.shape[1]
    hp, wp = H // 2, W // 2

    @pl.when(p == 0)
    def _phase0():
        @pl.when(i == 0)
        def _():
            st1_ref[...] = jnp.zeros_like(st1_ref)
            pad_ref[...] = jnp.zeros_like(pad_ref)
            slab_ref[:, wp:WPAD, :] = jnp.zeros(
                (hp, WPAD - wp, slab_ref.shape[2]), jnp.bfloat16)

        xs_ref[...] = jnp.transpose(x_ref[0]).reshape(H, W, cin)
        a = xs_ref[:, pl.ds(0, wp, stride=2), :]
        b = xs_ref[:, pl.ds(1, wp, stride=2), :]
        xw = jnp.maximum(a, b)
        pooled = jnp.max(xw.reshape(hp, 2, wp, cin), axis=1)
        pad_ref[1:hp + 1, COL0:COL0 + wp, 0:cin] = pooled.astype(jnp.bfloat16)

        acc = _build_slab_and_matmul(pad_ref, slab_ref, w1_ref, hp, wp)
        y_ref[i] = acc.reshape(hp, WPAD, LANE).astype(jnp.bfloat16)
        s = jnp.sum(acc, axis=0, keepdims=True)
        sq = jnp.sum(acc * acc, axis=0, keepdims=True)
        st1_ref[...] += jnp.concatenate([s, sq], axis=0)

    @pl.when(p == 1)
    def _phase1():
        @pl.when(i == 0)
        def _():
            st2_ref[...] = jnp.zeros_like(st2_ref)

        scale, shift = _scale_shift(st1_ref, g1_ref, b1_ref, count)
        h = jnp.maximum(
            y_ref[i][:, 0:wp, :].astype(jnp.float32) * scale + shift, 0.0)
        pad_ref[1:hp + 1, COL0:COL0 + wp, :] = h.astype(jnp.bfloat16)

        acc = _build_slab_and_matmul(pad_ref, slab_ref, w2_ref, hp, wp)
        y_ref[i] = acc.reshape(hp, WPAD, LANE).astype(jnp.bfloat16)
        s = jnp.sum(acc, axis=0, keepdims=True)
        sq = jnp.sum(acc * acc, axis=0, keepdims=True)
        st2_ref[...] += jnp.concatenate([s, sq], axis=0)

    @pl.when(p == 2)
    def _phase2():
        scale, shift = _scale_shift(st2_ref, g2_ref, b2_ref, count)
        h = jnp.maximum(
            y_ref[i][:, 0:wp, :].astype(jnp.float32) * scale + shift, 0.0)
        out_ref[0] = jnp.transpose(h.reshape(hp * wp, LANE))


def _pack_conv_w(w_oihw):
    co, ci, _, _ = w_oihw.shape
    w = jnp.transpose(w_oihw, (2, 3, 1, 0))
    w = jnp.pad(w, ((0, 0), (0, 0), (0, LANE - ci), (0, 0)))
    return w.reshape(9 * LANE, co).astype(jnp.bfloat16)


@jax.jit
def kernel(x_nchw, w1, b1, g1, be1, w2, b2, g2, be2):
    del b1, b2
    N, Cin, H, W = x_nchw.shape
    Hp, Wp = H // 2, W // 2
    Cout = w1.shape[0]

    x_flat = x_nchw.reshape(N, Cin, H * W)
    w1p = _pack_conv_w(w1)
    w2p = _pack_conv_w(w2)
    count = float(N * Hp * Wp)

    body = functools.partial(_down_kernel, count)

    out = pl.pallas_call(
        body,
        grid=(3, N),
        in_specs=[
            pl.BlockSpec((1, Cin, H * W), lambda p, i: ((p == 0) * i, 0, 0)),
            pl.BlockSpec((9 * LANE, Cout), lambda p, i: (0, 0)),
            pl.BlockSpec((9 * LANE, Cout), lambda p, i: (0, 0)),
            pl.BlockSpec((1, Cout), lambda p, i: (0, 0)),
            pl.BlockSpec((1, Cout), lambda p, i: (0, 0)),
            pl.BlockSpec((1, Cout), lambda p, i: (0, 0)),
            pl.BlockSpec((1, Cout), lambda p, i: (0, 0)),
        ],
        out_specs=pl.BlockSpec((1, Cout, Hp * Wp),
                               lambda p, i: ((p == 2) * i, 0, 0)),
        out_shape=jax.ShapeDtypeStruct((N, Cout, Hp * Wp), jnp.float32),
        scratch_shapes=[
            pltpu.VMEM((H, W, Cin), jnp.float32),
            pltpu.VMEM((N, Hp, WPAD, LANE), jnp.bfloat16),
            pltpu.VMEM((Hp + 2, COL0 + WPAD, LANE), jnp.bfloat16),
            pltpu.VMEM((Hp, WPAD, 9 * LANE), jnp.bfloat16),
            pltpu.VMEM((2, LANE), jnp.float32),
            pltpu.VMEM((2, LANE), jnp.float32),
        ],
        compiler_params=pltpu.CompilerParams(
            dimension_semantics=("arbitrary", "arbitrary"),
            vmem_limit_bytes=64 * 1024 * 1024),
    )(x_flat, w1p, w2p, g1.reshape(1, Cout), be1.reshape(1, Cout),
      g2.reshape(1, Cout), be2.reshape(1, Cout))

    return out.reshape(N, Cout, Hp, Wp)


# speedup vs baseline: 1.0379x; 1.0379x over previous
import functools

import jax
import jax.numpy as jnp
from jax.experimental import pallas as pl
from jax.experimental.pallas import tpu as pltpu

EPS = 1e-5
LANE = 128
WPAD = 64
COL0 = 16


def _build_slab_and_matmul(pad_ref, slab_ref, w_ref, hp, wp):
    for dh in range(3):
        for dw in range(3):
            t = dh * 3 + dw
            slab_ref[:, 0:wp, t * LANE:(t + 1) * LANE] = (
                pad_ref[dh:dh + hp, COL0 - 1 + dw:COL0 - 1 + dw + wp, :])
    return jnp.dot(slab_ref[...].reshape(hp * WPAD, 9 * LANE), w_ref[...],
                   preferred_element_type=jnp.float32)


def _scale_shift(st_ref, g_ref, b_ref, count):
    s = st_ref[0, :]
    sq = st_ref[1, :]
    mean = s / count
    var = jnp.maximum(sq / count - mean * mean, 0.0)
    scale = g_ref[0] * jax.lax.rsqrt(var + EPS)
    shift = b_ref[0] - mean * scale
    return scale, shift


def _down_kernel(count, x_ref, w1_ref, w2_ref, g1_ref, b1_ref, g2_ref,
                 b2_ref, out_ref, xs_ref, y_ref, pad_ref, slab_ref,
                 st1_ref, st2_ref):
    p = pl.program_id(0)
    i = pl.program_id(1)
    cin = x_ref.shape[1]
    H = xs_ref.shape[0]
    W = xs_ref.shape[1]
    hp, wp = H // 2, W // 2
    cout = out_ref.shape[1]

    @pl.when(p == 0)
    def _phase0():
        @pl.when(i == 0)
        def _():
            st1_ref[...] = jnp.zeros_like(st1_ref)
            pad_ref[...] = jnp.zeros_like(pad_ref)
            slab_ref[:, wp:WPAD, :] = jnp.zeros(
                (hp, WPAD - wp, slab_ref.shape[2]), jnp.bfloat16)

        xs_ref[...] = jnp.transpose(
            x_ref[0].reshape(cin, H * W)).reshape(H, W, cin)
        a = xs_ref[:, pl.ds(0, wp, stride=2), :]
        b = xs_ref[:, pl.ds(1, wp, stride=2), :]
        xw = jnp.maximum(a, b)
        pooled = jnp.max(xw.reshape(hp, 2, wp, cin), axis=1)
        pad_ref[1:hp + 1, COL0:COL0 + wp, 0:cin] = pooled.astype(jnp.bfloat16)

        acc = _build_slab_and_matmul(pad_ref, slab_ref, w1_ref, hp, wp)
        y_ref[i] = acc.reshape(hp, WPAD, LANE).astype(jnp.bfloat16)
        s = jnp.sum(acc, axis=0, keepdims=True)
        sq = jnp.sum(acc * acc, axis=0, keepdims=True)
        st1_ref[...] += jnp.concatenate([s, sq], axis=0)

    @pl.when(p == 1)
    def _phase1():
        @pl.when(i == 0)
        def _():
            st2_ref[...] = jnp.zeros_like(st2_ref)

        scale, shift = _scale_shift(st1_ref, g1_ref, b1_ref, count)
        h = jnp.maximum(
            y_ref[i][:, 0:wp, :].astype(jnp.float32) * scale + shift, 0.0)
        pad_ref[1:hp + 1, COL0:COL0 + wp, :] = h.astype(jnp.bfloat16)

        acc = _build_slab_and_matmul(pad_ref, slab_ref, w2_ref, hp, wp)
        y_ref[i] = acc.reshape(hp, WPAD, LANE).astype(jnp.bfloat16)
        s = jnp.sum(acc, axis=0, keepdims=True)
        sq = jnp.sum(acc * acc, axis=0, keepdims=True)
        st2_ref[...] += jnp.concatenate([s, sq], axis=0)

    @pl.when(p == 2)
    def _phase2():
        scale, shift = _scale_shift(st2_ref, g2_ref, b2_ref, count)
        h = jnp.maximum(
            y_ref[i][:, 0:wp, :].astype(jnp.float32) * scale + shift, 0.0)
        out_ref[0] = jnp.transpose(h.reshape(hp * wp, LANE)).reshape(
            cout, hp, wp)


def _pack_conv_w(w_oihw):
    co, ci, _, _ = w_oihw.shape
    w = jnp.transpose(w_oihw, (2, 3, 1, 0))
    w = jnp.pad(w, ((0, 0), (0, 0), (0, LANE - ci), (0, 0)))
    return w.reshape(9 * LANE, co).astype(jnp.bfloat16)


@jax.jit
def kernel(x_nchw, w1, b1, g1, be1, w2, b2, g2, be2):
    del b1, b2
    N, Cin, H, W = x_nchw.shape
    Hp, Wp = H // 2, W // 2
    Cout = w1.shape[0]

    w1p = _pack_conv_w(w1)
    w2p = _pack_conv_w(w2)
    count = float(N * Hp * Wp)

    body = functools.partial(_down_kernel, count)

    out = pl.pallas_call(
        body,
        grid=(3, N),
        in_specs=[
            pl.BlockSpec((1, Cin, H, W),
                         lambda p, i: ((p == 0) * i, 0, 0, 0)),
            pl.BlockSpec((9 * LANE, Cout), lambda p, i: (0, 0)),
            pl.BlockSpec((9 * LANE, Cout), lambda p, i: (0, 0)),
            pl.BlockSpec((1, Cout), lambda p, i: (0, 0)),
            pl.BlockSpec((1, Cout), lambda p, i: (0, 0)),
            pl.BlockSpec((1, Cout), lambda p, i: (0, 0)),
            pl.BlockSpec((1, Cout), lambda p, i: (0, 0)),
        ],
        out_specs=pl.BlockSpec((1, Cout, Hp, Wp),
                               lambda p, i: ((p == 2) * i, 0, 0, 0)),
        out_shape=jax.ShapeDtypeStruct((N, Cout, Hp, Wp), jnp.float32),
        scratch_shapes=[
            pltpu.VMEM((H, W, Cin), jnp.float32),
            pltpu.VMEM((N, Hp, WPAD, LANE), jnp.bfloat16),
            pltpu.VMEM((Hp + 2, COL0 + WPAD, LANE), jnp.bfloat16),
            pltpu.VMEM((Hp, WPAD, 9 * LANE), jnp.bfloat16),
            pltpu.VMEM((2, LANE), jnp.float32),
            pltpu.VMEM((2, LANE), jnp.float32),
        ],
        compiler_params=pltpu.CompilerParams(
            dimension_semantics=("arbitrary", "arbitrary"),
            vmem_limit_bytes=64 * 1024 * 1024),
    )(x_nchw, w1p, w2p, g1.reshape(1, Cout), be1.reshape(1, Cout),
      g2.reshape(1, Cout), be2.reshape(1, Cout))

    return out


# speedup vs baseline: 1.1954x; 1.1518x over previous
import functools

import jax
import jax.numpy as jnp
from jax.experimental import pallas as pl
from jax.experimental.pallas import tpu as pltpu

EPS = 1e-5
LANE = 128
WPAD = 64
PADW = 96


TAP_PAIRS = (((0, 0), (0, 1)), ((0, 2), (1, 0)), ((1, 1), (1, 2)),
             ((2, 0), (2, 1)), ((2, 2), (2, 2)))
MCH = 4


def _conv3x3(pads, w_ref, y_ref, i, st_ref, hp, wp):
    rows = hp // MCH
    m = rows * WPAD
    keep = jax.lax.broadcasted_iota(jnp.int32, (rows, WPAD, LANE), 1) < wp
    s = jnp.zeros((LANE,), jnp.float32)
    sq = jnp.zeros((LANE,), jnp.float32)
    for c in range(MCH):
        mxu = c % 2
        for j, ((dha, dwa), (dhb, dwb)) in enumerate(TAP_PAIRS):
            la = pads[dwa][dha + c * rows:dha + (c + 1) * rows,
                           16:16 + WPAD, :].reshape(m, LANE)
            lb = pads[dwb][dhb + c * rows:dhb + (c + 1) * rows,
                           16:16 + WPAD, :].reshape(m, LANE)
            pltpu.matmul_push_rhs(w_ref[j * 256:(j + 1) * 256, :],
                                  staging_register=j % 2, mxu_index=mxu)
            pltpu.matmul_acc_lhs(0, jnp.concatenate([la, lb], axis=-1),
                                 mxu_index=mxu, load_staged_rhs=j % 2)
        res = pltpu.matmul_pop(0, (m, 256), jnp.float32,
                               mxu_index=mxu)[:, 0:LANE]
        resm = jnp.where(keep, res.reshape(rows, WPAD, LANE), 0.0)
        y_ref[i, c * rows:(c + 1) * rows] = resm.astype(jnp.bfloat16)
        s = s + jnp.sum(resm, axis=(0, 1))
        sq = sq + jnp.sum(resm * resm, axis=(0, 1))
    st_ref[...] += jnp.stack([s, sq], axis=0)


def _store_interior(pads, h_bf16, hp, wp):
    pads[0][1:hp + 1, 17:17 + wp, :] = h_bf16
    pads[1][1:hp + 1, 16:16 + wp, :] = h_bf16
    pads[2][1:hp + 1, 15:15 + wp, :] = h_bf16


def _scale_shift(st_ref, g_ref, b_ref, count):
    s = st_ref[0, :]
    sq = st_ref[1, :]
    mean = s / count
    var = jnp.maximum(sq / count - mean * mean, 0.0)
    scale = g_ref[0] * jax.lax.rsqrt(var + EPS)
    shift = b_ref[0] - mean * scale
    return scale, shift


def _down_kernel(count, x_ref, w1_ref, w2_ref, g1_ref, b1_ref, g2_ref,
                 b2_ref, out_ref, xs_ref, y_ref, pad0_ref, pad1_ref,
                 pad2_ref, st1_ref, st2_ref):
    p = pl.program_id(0)
    i = pl.program_id(1)
    cin = x_ref.shape[1]
    H = xs_ref.shape[0]
    W = xs_ref.shape[1]
    hp, wp = H // 2, W // 2
    cout = out_ref.shape[1]
    pads = (pad0_ref, pad1_ref, pad2_ref)

    @pl.when(p == 0)
    def _phase0():
        @pl.when(i == 0)
        def _():
            st1_ref[...] = jnp.zeros_like(st1_ref)
            pad0_ref[...] = jnp.zeros_like(pad0_ref)
            pad1_ref[...] = jnp.zeros_like(pad1_ref)
            pad2_ref[...] = jnp.zeros_like(pad2_ref)

        xs_ref[...] = jnp.transpose(
            x_ref[0].reshape(cin, H * W)).reshape(H, W, cin)
        a = xs_ref[:, pl.ds(0, wp, stride=2), :]
        b = xs_ref[:, pl.ds(1, wp, stride=2), :]
        xw = jnp.maximum(a, b)
        pooled = jnp.max(xw.reshape(hp, 2, wp, cin), axis=1)
        pb = jnp.pad(pooled.astype(jnp.bfloat16),
                     ((0, 0), (0, 0), (0, LANE - cin)))
        _store_interior(pads, pb, hp, wp)

        _conv3x3(pads, w1_ref, y_ref, i, st1_ref, hp, wp)

    @pl.when(p == 1)
    def _phase1():
        @pl.when(i == 0)
        def _():
            st2_ref[...] = jnp.zeros_like(st2_ref)

        scale, shift = _scale_shift(st1_ref, g1_ref, b1_ref, count)
        h = jnp.maximum(
            y_ref[i][:, 0:wp, :].astype(jnp.float32) * scale + shift, 0.0)
        _store_interior(pads, h.astype(jnp.bfloat16), hp, wp)

        _conv3x3(pads, w2_ref, y_ref, i, st2_ref, hp, wp)

    @pl.when(p == 2)
    def _phase2():
        scale, shift = _scale_shift(st2_ref, g2_ref, b2_ref, count)
        h = jnp.maximum(
            y_ref[i][:, 0:wp, :].astype(jnp.float32) * scale + shift, 0.0)
        out_ref[0] = jnp.transpose(h.reshape(hp * wp, LANE)).reshape(
            cout, hp, wp)


def _pack_conv_w(w_oihw):
    co, ci, _, _ = w_oihw.shape
    w = jnp.transpose(w_oihw, (2, 3, 1, 0))
    w = jnp.pad(w, ((0, 0), (0, 0), (0, LANE - ci), (0, 0)))
    zero = jnp.zeros((LANE, co), w.dtype)
    tiles = []
    for j, ((dha, dwa), (dhb, dwb)) in enumerate(TAP_PAIRS):
        wa = w[dha, dwa]
        wb = zero if j == len(TAP_PAIRS) - 1 else w[dhb, dwb]
        tiles.append(jnp.concatenate([wa, wb], axis=0))
    wk = jnp.concatenate(tiles, axis=0)
    wk = jnp.pad(wk, ((0, 0), (0, 256 - co)))
    return wk.astype(jnp.bfloat16)


@jax.jit
def kernel(x_nchw, w1, b1, g1, be1, w2, b2, g2, be2):
    del b1, b2
    N, Cin, H, W = x_nchw.shape
    Hp, Wp = H // 2, W // 2
    Cout = w1.shape[0]

    w1p = _pack_conv_w(w1)
    w2p = _pack_conv_w(w2)
    count = float(N * Hp * Wp)

    body = functools.partial(_down_kernel, count)

    out = pl.pallas_call(
        body,
        grid=(3, N),
        in_specs=[
            pl.BlockSpec((1, Cin, H, W),
                         lambda p, i: ((p == 0) * i, 0, 0, 0)),
            pl.BlockSpec((5 * 256, 256), lambda p, i: (0, 0)),
            pl.BlockSpec((5 * 256, 256), lambda p, i: (0, 0)),
            pl.BlockSpec((1, Cout), lambda p, i: (0, 0)),
            pl.BlockSpec((1, Cout), lambda p, i: (0, 0)),
            pl.BlockSpec((1, Cout), lambda p, i: (0, 0)),
            pl.BlockSpec((1, Cout), lambda p, i: (0, 0)),
        ],
        out_specs=pl.BlockSpec((1, Cout, Hp, Wp),
                               lambda p, i: ((p == 2) * i, 0, 0, 0)),
        out_shape=jax.ShapeDtypeStruct((N, Cout, Hp, Wp), jnp.float32),
        scratch_shapes=[
            pltpu.VMEM((H, W, Cin), jnp.float32),
            pltpu.VMEM((N, Hp, WPAD, LANE), jnp.bfloat16),
            pltpu.VMEM((Hp + 2, PADW, LANE), jnp.bfloat16),
            pltpu.VMEM((Hp + 2, PADW, LANE), jnp.bfloat16),
            pltpu.VMEM((Hp + 2, PADW, LANE), jnp.bfloat16),
            pltpu.VMEM((2, LANE), jnp.float32),
            pltpu.VMEM((2, LANE), jnp.float32),
        ],
        compiler_params=pltpu.CompilerParams(
            dimension_semantics=("arbitrary", "arbitrary"),
            vmem_limit_bytes=64 * 1024 * 1024),
    )(x_nchw, w1p, w2p, g1.reshape(1, Cout), be1.reshape(1, Cout),
      g2.reshape(1, Cout), be2.reshape(1, Cout))

    return out


# speedup vs baseline: 1.2693x; 1.0618x over previous
import functools

import jax
import jax.numpy as jnp
from jax.experimental import pallas as pl
from jax.experimental.pallas import tpu as pltpu

EPS = 1e-5
LANE = 128
WPAD = 64
PADW = 96


TAP_PAIRS = (((0, 0), (0, 1)), ((0, 2), (1, 0)), ((1, 1), (1, 2)),
             ((2, 0), (2, 1)), ((2, 2), (2, 2)))
MCH = 4
TILE_N = 2


def _conv3x3(pads, w_ref, y_ref, i, st_ref, hp, wp):
    rows = hp // MCH
    m = rows * WPAD
    s = jnp.zeros((LANE,), jnp.float32)
    sq = jnp.zeros((LANE,), jnp.float32)
    for c in range(MCH):
        mxu = c % 2
        for j, ((dha, dwa), (dhb, dwb)) in enumerate(TAP_PAIRS):
            la = pads[dwa][dha + c * rows:dha + (c + 1) * rows,
                           16:16 + WPAD, :].reshape(m, LANE)
            lb = pads[dwb][dhb + c * rows:dhb + (c + 1) * rows,
                           16:16 + WPAD, :].reshape(m, LANE)
            pltpu.matmul_push_rhs(w_ref[j * 256:(j + 1) * 256, :],
                                  staging_register=j % 2, mxu_index=mxu)
            pltpu.matmul_acc_lhs(0, jnp.concatenate([la, lb], axis=-1),
                                 mxu_index=mxu, load_staged_rhs=j % 2)
        res = pltpu.matmul_pop(0, (m, 256), jnp.float32,
                               mxu_index=mxu)[:, 0:LANE]
        resm = res.reshape(rows, WPAD, LANE)
        y_ref[i, c * rows:(c + 1) * rows] = resm.astype(jnp.bfloat16)
        s = s + jnp.sum(resm, axis=(0, 1))
        sq = sq + jnp.sum(resm * resm, axis=(0, 1))
    st_ref[...] += jnp.stack([s, sq], axis=0)


def _store_interior(pads, h_bf16, hp, wp):
    pads[0][1:hp + 1, 17:16 + wp, :] = h_bf16[:, 0:wp - 1, :]
    pads[1][1:hp + 1, 16:16 + wp, :] = h_bf16
    pads[2][1:hp + 1, 15:15 + wp, :] = h_bf16


def _scale_shift(st_ref, g_ref, b_ref, count):
    s = st_ref[0, :]
    sq = st_ref[1, :]
    mean = s / count
    var = jnp.maximum(sq / count - mean * mean, 0.0)
    scale = g_ref[0] * jax.lax.rsqrt(var + EPS)
    shift = b_ref[0] - mean * scale
    return scale, shift


def _down_kernel(count, x_ref, w1_ref, w2_ref, g1_ref, b1_ref, g2_ref,
                 b2_ref, out_ref, xs_ref, y_ref, pad0_ref, pad1_ref,
                 pad2_ref, st1_ref, st2_ref):
    p = pl.program_id(0)
    i = pl.program_id(1)
    cin = x_ref.shape[1]
    H = xs_ref.shape[0]
    W = xs_ref.shape[1]
    hp, wp = H // 2, W // 2
    cout = out_ref.shape[1]
    pads = (pad0_ref, pad1_ref, pad2_ref)

    @pl.when(p == 0)
    def _phase0():
        @pl.when(i == 0)
        def _():
            st1_ref[...] = jnp.zeros_like(st1_ref)
            pad0_ref[...] = jnp.zeros_like(pad0_ref)
            pad1_ref[...] = jnp.zeros_like(pad1_ref)
            pad2_ref[...] = jnp.zeros_like(pad2_ref)

        for img in range(TILE_N):
            xs_ref[...] = jnp.transpose(
                x_ref[img].reshape(cin, H * W)).reshape(H, W, cin)
            a = xs_ref[:, pl.ds(0, wp, stride=2), :]
            b = xs_ref[:, pl.ds(1, wp, stride=2), :]
            xw = jnp.maximum(a, b)
            pooled = jnp.max(xw.reshape(hp, 2, wp, cin), axis=1)
            pb = jnp.pad(pooled.astype(jnp.bfloat16),
                         ((0, 0), (0, 0), (0, LANE - cin)))
            _store_interior(pads, pb, hp, wp)
            _conv3x3(pads, w1_ref, y_ref, i * TILE_N + img, st1_ref, hp, wp)

    @pl.when(p == 1)
    def _phase1():
        @pl.when(i == 0)
        def _():
            st2_ref[...] = jnp.zeros_like(st2_ref)

        scale, shift = _scale_shift(st1_ref, g1_ref, b1_ref, count)
        for img in range(TILE_N):
            h = jnp.maximum(
                y_ref[i * TILE_N + img][:, 0:wp, :].astype(jnp.float32)
                * scale + shift, 0.0)
            _store_interior(pads, h.astype(jnp.bfloat16), hp, wp)
            _conv3x3(pads, w2_ref, y_ref, i * TILE_N + img, st2_ref, hp, wp)

    @pl.when(p == 2)
    def _phase2():
        scale, shift = _scale_shift(st2_ref, g2_ref, b2_ref, count)
        for img in range(TILE_N):
            h = jnp.maximum(
                y_ref[i * TILE_N + img][:, 0:wp, :].astype(jnp.float32)
                * scale + shift, 0.0)
            out_ref[img] = jnp.transpose(h.reshape(hp * wp, LANE)).reshape(
                cout, hp, wp)


def _pack_conv_w(w_oihw):
    co, ci, _, _ = w_oihw.shape
    w = jnp.transpose(w_oihw, (2, 3, 1, 0))
    w = jnp.pad(w, ((0, 0), (0, 0), (0, LANE - ci), (0, 0)))
    zero = jnp.zeros((LANE, co), w.dtype)
    tiles = []
    for j, ((dha, dwa), (dhb, dwb)) in enumerate(TAP_PAIRS):
        wa = w[dha, dwa]
        wb = zero if j == len(TAP_PAIRS) - 1 else w[dhb, dwb]
        tiles.append(jnp.concatenate([wa, wb], axis=0))
    wk = jnp.concatenate(tiles, axis=0)
    wk = jnp.pad(wk, ((0, 0), (0, 256 - co)))
    return wk.astype(jnp.bfloat16)


@jax.jit
def kernel(x_nchw, w1, b1, g1, be1, w2, b2, g2, be2):
    del b1, b2
    N, Cin, H, W = x_nchw.shape
    Hp, Wp = H // 2, W // 2
    Cout = w1.shape[0]

    w1p = _pack_conv_w(w1)
    w2p = _pack_conv_w(w2)
    count = float(N * Hp * Wp)

    body = functools.partial(_down_kernel, count)

    out = pl.pallas_call(
        body,
        grid=(3, N // TILE_N),
        in_specs=[
            pl.BlockSpec((TILE_N, Cin, H, W),
                         lambda p, i: ((p == 0) * i, 0, 0, 0)),
            pl.BlockSpec((5 * 256, 256), lambda p, i: (0, 0)),
            pl.BlockSpec((5 * 256, 256), lambda p, i: (0, 0)),
            pl.BlockSpec((1, Cout), lambda p, i: (0, 0)),
            pl.BlockSpec((1, Cout), lambda p, i: (0, 0)),
            pl.BlockSpec((1, Cout), lambda p, i: (0, 0)),
            pl.BlockSpec((1, Cout), lambda p, i: (0, 0)),
        ],
        out_specs=pl.BlockSpec((TILE_N, Cout, Hp, Wp),
                               lambda p, i: ((p == 2) * i, 0, 0, 0)),
        out_shape=jax.ShapeDtypeStruct((N, Cout, Hp, Wp), jnp.float32),
        scratch_shapes=[
            pltpu.VMEM((H, W, Cin), jnp.float32),
            pltpu.VMEM((N, Hp, WPAD, LANE), jnp.bfloat16),
            pltpu.VMEM((Hp + 2, PADW, LANE), jnp.bfloat16),
            pltpu.VMEM((Hp + 2, PADW, LANE), jnp.bfloat16),
            pltpu.VMEM((Hp + 2, PADW, LANE), jnp.bfloat16),
            pltpu.VMEM((2, LANE), jnp.float32),
            pltpu.VMEM((2, LANE), jnp.float32),
        ],
        compiler_params=pltpu.CompilerParams(
            dimension_semantics=("arbitrary", "arbitrary"),
            vmem_limit_bytes=64 * 1024 * 1024),
    )(x_nchw, w1p, w2p, g1.reshape(1, Cout), be1.reshape(1, Cout),
      g2.reshape(1, Cout), be2.reshape(1, Cout))

    return out


# speedup vs baseline: 1.8782x; 1.4797x over previous
import functools

import jax
import jax.numpy as jnp
from jax.experimental import pallas as pl
from jax.experimental.pallas import tpu as pltpu

EPS = 1e-5
LANE = 128
WPAD = 64
PADW = 96


TAP_PAIRS = (((0, 0), (0, 1)), ((0, 2), (1, 0)), ((1, 1), (1, 2)),
             ((2, 0), (2, 1)), ((2, 2), (2, 2)))
MCH = 4
TILE_N = 2


def _conv3x3(pads, w_ref, y_ref, i, st_ref, hp, wp):
    rows = hp // MCH
    m = rows * WPAD
    s = jnp.zeros((LANE,), jnp.float32)
    sq = jnp.zeros((LANE,), jnp.float32)
    for c in range(MCH):
        mxu = c % 2
        for j, ((dha, dwa), (dhb, dwb)) in enumerate(TAP_PAIRS):
            la = pads[dwa][dha + c * rows:dha + (c + 1) * rows,
                           16:16 + WPAD, :].reshape(m, LANE)
            lb = pads[dwb][dhb + c * rows:dhb + (c + 1) * rows,
                           16:16 + WPAD, :].reshape(m, LANE)
            pltpu.matmul_push_rhs(w_ref[j * 256:(j + 1) * 256, :],
                                  staging_register=j % 2, mxu_index=mxu)
            pltpu.matmul_acc_lhs(0, jnp.concatenate([la, lb], axis=-1),
                                 mxu_index=mxu, load_staged_rhs=j % 2)
        res = pltpu.matmul_pop(0, (m, 256), jnp.float32,
                               mxu_index=mxu)[:, 0:LANE]
        resm = res.reshape(rows, WPAD, LANE)
        y_ref[i, c * rows:(c + 1) * rows] = resm.astype(jnp.bfloat16)
        s = s + jnp.sum(resm, axis=(0, 1))
        sq = sq + jnp.sum(resm * resm, axis=(0, 1))
    st_ref[...] += jnp.stack([s, sq], axis=0)


def _store_interior(pads, h_bf16, hp, wp):
    pads[0][1:hp + 1, 17:16 + wp, :] = h_bf16[:, 0:wp - 1, :]
    pads[1][1:hp + 1, 16:16 + wp, :] = h_bf16
    pads[2][1:hp + 1, 15:15 + wp, :] = h_bf16


def _scale_shift(st_ref, g_ref, b_ref, count):
    s = st_ref[0, :]
    sq = st_ref[1, :]
    mean = s / count
    var = jnp.maximum(sq / count - mean * mean, 0.0)
    scale = g_ref[0] * jax.lax.rsqrt(var + EPS)
    shift = b_ref[0] - mean * scale
    return scale, shift


def _down_kernel(count, x_ref, w1_ref, w2_ref, g1_ref, b1_ref, g2_ref,
                 b2_ref, out_ref, xs_ref, y_ref, pad0_ref, pad1_ref,
                 pad2_ref, st1_ref, st2_ref):
    p = pl.program_id(0)
    i = pl.program_id(1)
    cin = x_ref.shape[1]
    H = xs_ref.shape[0]
    W = xs_ref.shape[1]
    hp, wp = H // 2, W // 2
    pads = (pad0_ref, pad1_ref, pad2_ref)

    @pl.when(p == 0)
    def _phase0():
        @pl.when(i == 0)
        def _():
            st1_ref[...] = jnp.zeros_like(st1_ref)
            pad0_ref[...] = jnp.zeros_like(pad0_ref)
            pad1_ref[...] = jnp.zeros_like(pad1_ref)
            pad2_ref[...] = jnp.zeros_like(pad2_ref)

        for img in range(TILE_N):
            xs_ref[...] = jnp.transpose(
                x_ref[img].reshape(cin, H * W)).reshape(H, W, cin)
            a = xs_ref[:, pl.ds(0, wp, stride=2), :]
            b = xs_ref[:, pl.ds(1, wp, stride=2), :]
            xw = jnp.maximum(a, b)
            pooled = jnp.max(xw.reshape(hp, 2, wp, cin), axis=1)
            pb = jnp.pad(pooled.astype(jnp.bfloat16),
                         ((0, 0), (0, 0), (0, LANE - cin)))
            _store_interior(pads, pb, hp, wp)
            _conv3x3(pads, w1_ref, y_ref, i * TILE_N + img, st1_ref, hp, wp)

    @pl.when(p == 1)
    def _phase1():
        @pl.when(i == 0)
        def _():
            st2_ref[...] = jnp.zeros_like(st2_ref)

        scale, shift = _scale_shift(st1_ref, g1_ref, b1_ref, count)
        for img in range(TILE_N):
            h = jnp.maximum(
                y_ref[i * TILE_N + img][:, 0:wp, :].astype(jnp.float32)
                * scale + shift, 0.0)
            _store_interior(pads, h.astype(jnp.bfloat16), hp, wp)
            _conv3x3(pads, w2_ref, y_ref, i * TILE_N + img, st2_ref, hp, wp)

    @pl.when(p == 2)
    def _phase2():
        scale, shift = _scale_shift(st2_ref, g2_ref, b2_ref, count)
        for img in range(TILE_N):
            h = jnp.maximum(
                y_ref[i * TILE_N + img][:, 0:wp, :].astype(jnp.float32)
                * scale + shift, 0.0)
            out_ref[img] = h


def _pack_conv_w(w_oihw):
    co, ci, _, _ = w_oihw.shape
    w = jnp.transpose(w_oihw, (2, 3, 1, 0))
    w = jnp.pad(w, ((0, 0), (0, 0), (0, LANE - ci), (0, 0)))
    zero = jnp.zeros((LANE, co), w.dtype)
    tiles = []
    for j, ((dha, dwa), (dhb, dwb)) in enumerate(TAP_PAIRS):
        wa = w[dha, dwa]
        wb = zero if j == len(TAP_PAIRS) - 1 else w[dhb, dwb]
        tiles.append(jnp.concatenate([wa, wb], axis=0))
    wk = jnp.concatenate(tiles, axis=0)
    wk = jnp.pad(wk, ((0, 0), (0, 256 - co)))
    return wk.astype(jnp.bfloat16)


@jax.jit
def kernel(x_nchw, w1, b1, g1, be1, w2, b2, g2, be2):
    del b1, b2
    N, Cin, H, W = x_nchw.shape
    Hp, Wp = H // 2, W // 2
    Cout = w1.shape[0]

    w1p = _pack_conv_w(w1)
    w2p = _pack_conv_w(w2)
    count = float(N * Hp * Wp)

    body = functools.partial(_down_kernel, count)

    out = pl.pallas_call(
        body,
        grid=(3, N // TILE_N),
        in_specs=[
            pl.BlockSpec((TILE_N, Cin, H, W),
                         lambda p, i: ((p == 0) * i, 0, 0, 0)),
            pl.BlockSpec((5 * 256, 256), lambda p, i: (0, 0)),
            pl.BlockSpec((5 * 256, 256), lambda p, i: (0, 0)),
            pl.BlockSpec((1, Cout), lambda p, i: (0, 0)),
            pl.BlockSpec((1, Cout), lambda p, i: (0, 0)),
            pl.BlockSpec((1, Cout), lambda p, i: (0, 0)),
            pl.BlockSpec((1, Cout), lambda p, i: (0, 0)),
        ],
        out_specs=pl.BlockSpec((TILE_N, Hp, Wp, Cout),
                               lambda p, i: ((p == 2) * i, 0, 0, 0)),
        out_shape=jax.ShapeDtypeStruct((N, Hp, Wp, Cout), jnp.float32),
        scratch_shapes=[
            pltpu.VMEM((H, W, Cin), jnp.float32),
            pltpu.VMEM((N, Hp, WPAD, LANE), jnp.bfloat16),
            pltpu.VMEM((Hp + 2, PADW, LANE), jnp.bfloat16),
            pltpu.VMEM((Hp + 2, PADW, LANE), jnp.bfloat16),
            pltpu.VMEM((Hp + 2, PADW, LANE), jnp.bfloat16),
            pltpu.VMEM((2, LANE), jnp.float32),
            pltpu.VMEM((2, LANE), jnp.float32),
        ],
        compiler_params=pltpu.CompilerParams(
            dimension_semantics=("arbitrary", "arbitrary"),
            vmem_limit_bytes=64 * 1024 * 1024),
    )(x_nchw, w1p, w2p, g1.reshape(1, Cout), be1.reshape(1, Cout),
      g2.reshape(1, Cout), be2.reshape(1, Cout))

    return jnp.transpose(out, (0, 3, 1, 2))
